# baseline (speedup 1.0000x reference)
import jax
import jax.numpy as jnp
from jax.experimental import pallas as pl
from jax.experimental.pallas import tpu as pltpu

N_UP = 16
N_DOWN = 16
N_ELEC = 32
N_ATOMS = 16
N_NBR = N_ELEC + N_ATOMS
BASIS = 4
KER = 8
EMBED = 16
LAYERS = 2
BATCH = 4096

B_BLK = 128


def _jastrow_kernel(xs_ref, mask_ref, hn_ref, we_ref, be_ref, wn_ref, bn_ref,
                    wh_ref, bh_ref, wg_ref, bg_ref, wo_ref, bo_ref, ee_ref,
                    out_ref):
    B = B_BLK
    X = xs_ref[...]
    mask = mask_ref[...]

    x = []
    for e in range(EMBED):
        up = jnp.full((N_UP, B), ee_ref[0, e], dtype=jnp.float32)
        dn = jnp.full((N_DOWN, B), ee_ref[1, e], dtype=jnp.float32)
        x.append(jnp.concatenate([up, dn], axis=0))

    for l in range(LAYERS):
        h = []
        for k in range(KER):
            acc = jnp.full((N_ELEC, B), bh_ref[l, k], dtype=jnp.float32)
            for e in range(EMBED):
                acc = acc + x[e] * wh_ref[l, e, k]
            h.append(jnp.tanh(acc))

        z = []
        for k in range(KER):
            pe = jnp.full((N_ELEC, N_ELEC, B), be_ref[l, k], dtype=jnp.float32)
            for f in range(BASIS):
                pe = pe + X[:, :N_ELEC, f, :] * we_ref[l, f, k]
            we = jnp.tanh(pe) * mask
            zk = (we * h[k][None, :, :]).sum(axis=1)

            pn = jnp.full((N_ELEC, N_ATOMS, B), bn_ref[l, k], dtype=jnp.float32)
            for f in range(BASIS):
                pn = pn + X[:, N_ELEC:, f, :] * wn_ref[l, f, k]
            zk = zk + (jnp.tanh(pn) * hn_ref[k][None, :, :]).sum(axis=1)
            z.append(zk)

        for e in range(EMBED):
            acc = jnp.full((N_ELEC, B), bg_ref[l, e], dtype=jnp.float32)
            for k in range(KER):
                acc = acc + z[k] * wg_ref[l, k, e]
            x[e] = x[e] + jnp.tanh(acc)

    acc = jnp.full((N_ELEC, B), jnp.float32(N_ELEC) * bo_ref[0], jnp.float32)
    for e in range(EMBED):
        acc = acc + x[e] * wo_ref[e, 0]
    out_ref[...] = acc.sum(axis=0, keepdims=True)


@jax.jit
def kernel(xs, elec_embed, nuc_embed, Ww_e, bw_e, Ww_n, bw_n, Wh, bh, Wg, bg,
           Wo, bo):
    f32 = jnp.float32
    xs_n = xs.transpose(1, 2, 3, 0)

    mask = jnp.broadcast_to(
        (1.0 - jnp.eye(N_ELEC, dtype=f32))[:, :, None], (N_ELEC, N_ELEC, B_BLK))

    hn = jnp.broadcast_to(nuc_embed.T[:, :, None], (KER, N_ATOMS, B_BLK))

    grid = (BATCH // B_BLK,)

    def vmem_whole(shape):
        nd = len(shape)
        return pl.BlockSpec(shape, lambda i: (0,) * nd)

    def smem(shape):
        nd = len(shape)
        return pl.BlockSpec(shape, lambda i: (0,) * nd,
                            memory_space=pltpu.SMEM)

    out = pl.pallas_call(
        _jastrow_kernel,
        grid=grid,
        in_specs=[
            pl.BlockSpec((N_ELEC, N_NBR, BASIS, B_BLK),
                         lambda i: (0, 0, 0, i)),
            vmem_whole(mask.shape),
            vmem_whole(hn.shape),
            smem(Ww_e.shape),
            smem(bw_e.shape),
            smem(Ww_n.shape),
            smem(bw_n.shape),
            smem(Wh.shape),
            smem(bh.shape),
            smem(Wg.shape),
            smem(bg.shape),
            smem(Wo.shape),
            smem(bo.shape),
            smem(elec_embed.shape),
        ],
        out_specs=pl.BlockSpec((1, B_BLK), lambda i: (0, i)),
        out_shape=jax.ShapeDtypeStruct((1, BATCH), f32),
    )(xs_n, mask, hn, Ww_e, bw_e, Ww_n, bw_n, Wh, bh, Wg, bg, Wo, bo,
      elec_embed)
    return out.reshape(BATCH)


# speedup vs baseline: 1.0018x; 1.0018x over previous
import jax
import jax.numpy as jnp
from jax.experimental import pallas as pl
from jax.experimental.pallas import tpu as pltpu

N_UP = 16
N_DOWN = 16
N_ELEC = 32
N_ATOMS = 16
N_NBR = N_ELEC + N_ATOMS
BASIS = 4
KER = 8
EMBED = 16
LAYERS = 2
BATCH = 4096

B_BLK = 128


def _jastrow_kernel(xs_ref, mask_ref, hn_ref, we_ref, be_ref, wn_ref, bn_ref,
                    wh_ref, bh_ref, wg_ref, bg_ref, wo_ref, bo_ref, ee_ref,
                    out_ref):
    B = B_BLK
    X = xs_ref[...]
    mask = mask_ref[...]

    Xe = [X[:, :N_ELEC, f, :] for f in range(BASIS)]
    Xn = [X[:, N_ELEC:, f, :] for f in range(BASIS)]

    x = []
    for e in range(EMBED):
        up = jnp.full((N_UP, B), ee_ref[0, e], dtype=jnp.float32)
        dn = jnp.full((N_DOWN, B), ee_ref[1, e], dtype=jnp.float32)
        x.append(jnp.concatenate([up, dn], axis=0))

    for l in range(LAYERS):
        h = []
        for k in range(KER):
            acc = jnp.full((N_ELEC, B), bh_ref[l, k], dtype=jnp.float32)
            for e in range(EMBED):
                acc = acc + x[e] * wh_ref[l, e, k]
            h.append(jnp.tanh(acc))

        z = []
        for k in range(KER):
            pe = jnp.full((N_ELEC, N_ELEC, B), be_ref[l, k], dtype=jnp.float32)
            for f in range(BASIS):
                pe = pe + Xe[f] * we_ref[l, f, k]
            we = jnp.tanh(pe) * mask
            zk = (we * h[k][None, :, :]).sum(axis=1)

            pn = jnp.full((N_ELEC, N_ATOMS, B), bn_ref[l, k], dtype=jnp.float32)
            for f in range(BASIS):
                pn = pn + Xn[f] * wn_ref[l, f, k]
            zk = zk + (jnp.tanh(pn) * hn_ref[k][None, :, :]).sum(axis=1)
            z.append(zk)

        for e in range(EMBED):
            acc = jnp.full((N_ELEC, B), bg_ref[l, e], dtype=jnp.float32)
            for k in range(KER):
                acc = acc + z[k] * wg_ref[l, k, e]
            x[e] = x[e] + jnp.tanh(acc)

    acc = jnp.full((N_ELEC, B), jnp.float32(N_ELEC) * bo_ref[0], jnp.float32)
    for e in range(EMBED):
        acc = acc + x[e] * wo_ref[e, 0]
    out_ref[...] = acc.sum(axis=0, keepdims=True)


@jax.jit
def kernel(xs, elec_embed, nuc_embed, Ww_e, bw_e, Ww_n, bw_n, Wh, bh, Wg, bg,
           Wo, bo):
    f32 = jnp.float32
    xs_n = xs.transpose(1, 2, 3, 0)

    mask = jnp.broadcast_to(
        (1.0 - jnp.eye(N_ELEC, dtype=f32))[:, :, None], (N_ELEC, N_ELEC, B_BLK))

    hn = jnp.broadcast_to(nuc_embed.T[:, :, None], (KER, N_ATOMS, B_BLK))

    grid = (BATCH // B_BLK,)

    def vmem_whole(shape):
        nd = len(shape)
        return pl.BlockSpec(shape, lambda i: (0,) * nd)

    def smem(shape):
        nd = len(shape)
        return pl.BlockSpec(shape, lambda i: (0,) * nd,
                            memory_space=pltpu.SMEM)

    out = pl.pallas_call(
        _jastrow_kernel,
        grid=grid,
        in_specs=[
            pl.BlockSpec((N_ELEC, N_NBR, BASIS, B_BLK),
                         lambda i: (0, 0, 0, i)),
            vmem_whole(mask.shape),
            vmem_whole(hn.shape),
            smem(Ww_e.shape),
            smem(bw_e.shape),
            smem(Ww_n.shape),
            smem(bw_n.shape),
            smem(Wh.shape),
            smem(bh.shape),
            smem(Wg.shape),
            smem(bg.shape),
            smem(Wo.shape),
            smem(bo.shape),
            smem(elec_embed.shape),
        ],
        out_specs=pl.BlockSpec((1, B_BLK), lambda i: (0, i)),
        out_shape=jax.ShapeDtypeStruct((1, BATCH), f32),
    )(xs_n, mask, hn, Ww_e, bw_e, Ww_n, bw_n, Wh, bh, Wg, bg, Wo, bo,
      elec_embed)
    return out.reshape(BATCH)


# speedup vs baseline: 4.7084x; 4.7000x over previous
import jax
import jax.numpy as jnp
from jax.experimental import pallas as pl

N_UP = 16
N_DOWN = 16
N_ELEC = 32
N_ATOMS = 16
N_NBR = N_ELEC + N_ATOMS
BASIS = 4
KER = 8
EMBED = 16
LAYERS = 2
BATCH = 4096
LE = N_ELEC * KER
LN = N_ATOMS * KER

B_BLK = 128


def _jastrow_kernel(xs_ref, wcat_ref, bcat_ref, m0h_ref, mask_ref, diag_ref,
                    wht_ref, bht_ref, sele_ref, seln_ref, wg_ref, bg_ref,
                    wo_ref, x0_ref, bo_ref, out_ref):
    B = B_BLK
    R = B * N_ELEC
    bf16 = jnp.bfloat16
    f32 = jnp.float32
    X = xs_ref[...].astype(bf16)

    P = jnp.tanh(
        jax.lax.dot(X, wcat_ref[...], preferred_element_type=f32)
        + bcat_ref[...])
    P0e = P[:, :LE]
    P0n = P[:, LE:LE + LN]
    P1e = P[:, LE + LN:2 * LE + LN]
    P1n = P[:, 2 * LE + LN:]

    sele = sele_ref[...]
    seln = seln_ref[...]

    M0 = (P0e.reshape(B, N_ELEC, LE) * m0h_ref[...][None]).reshape(R, LE)
    z0 = (jax.lax.dot(M0.astype(bf16), sele, preferred_element_type=f32)
          + jax.lax.dot(P0n.astype(bf16), seln, preferred_element_type=f32))
    g0 = jnp.tanh(
        jax.lax.dot(z0.astype(bf16), wg_ref[0], preferred_element_type=f32)
        + bg_ref[0][None, :])
    x1 = jnp.broadcast_to(x0_ref[...][None], (B, N_ELEC, EMBED)).reshape(
        R, EMBED) + g0

    ph = (jax.lax.dot(x1.astype(bf16), wht_ref[...], preferred_element_type=f32)
          + bht_ref[...])
    hb = jnp.tanh((ph.reshape(B, N_ELEC, LE) * diag_ref[...][None]).sum(axis=1))
    M1 = (P1e.reshape(B, N_ELEC, LE) * mask_ref[...][None]
          * hb[:, None, :]).reshape(R, LE)
    z1 = (jax.lax.dot(M1.astype(bf16), sele, preferred_element_type=f32)
          + jax.lax.dot(P1n.astype(bf16), seln, preferred_element_type=f32))
    x2 = x1 + jnp.tanh(
        jax.lax.dot(z1.astype(bf16), wg_ref[1], preferred_element_type=f32)
        + bg_ref[1][None, :])

    t = (x2 * wo_ref[...]).sum(axis=1, keepdims=True)
    out_ref[...] = (t.reshape(B, N_ELEC, 1).sum(axis=1)
                    + jnp.float32(N_ELEC) * bo_ref[0, 0])


@jax.jit
def kernel(xs, elec_embed, nuc_embed, Ww_e, bw_e, Ww_n, bw_n, Wh, bh, Wg, bg,
           Wo, bo):
    f32 = jnp.float32
    bf16 = jnp.bfloat16

    scale = jnp.where(jnp.isfinite(bo[0]), f32(1.0), f32(0.0))
    xs_flat = xs.reshape(BATCH * N_ELEC, N_NBR * BASIS) * scale

    eye48 = jnp.eye(N_NBR, dtype=f32)
    w_sel = jnp.concatenate(
        [jnp.broadcast_to(Ww_e[:, None], (LAYERS, N_ELEC, BASIS, KER)),
         jnp.broadcast_to(Ww_n[:, None], (LAYERS, N_ATOMS, BASIS, KER))],
        axis=1)
    w_big = jnp.einsum('jJ,ljfk->ljfJk', eye48, w_sel).reshape(
        LAYERS, N_NBR * BASIS, N_NBR * KER)
    w_cat = jnp.concatenate([w_big[0], w_big[1]], axis=1).astype(bf16)
    b_big = jnp.concatenate(
        [jnp.broadcast_to(bw_e[:, None], (LAYERS, N_ELEC, KER)),
         jnp.broadcast_to(bw_n[:, None], (LAYERS, N_ATOMS, KER))],
        axis=1).reshape(LAYERS, N_NBR * KER)
    b_cat = jnp.concatenate([b_big[0], b_big[1]]).reshape(1, 2 * N_NBR * KER)

    eye_jk = jnp.repeat(jnp.eye(N_ELEC, dtype=f32), KER, axis=1)
    mask = 1.0 - eye_jk
    diag = eye_jk

    x0 = jnp.concatenate([
        jnp.broadcast_to(elec_embed[0][None], (N_UP, EMBED)),
        jnp.broadcast_to(elec_embed[1][None], (N_DOWN, EMBED)),
    ], axis=0)
    h0 = jnp.tanh(x0 @ Wh[0] + bh[0][None, :]).reshape(1, LE)
    m0h = mask * h0

    sel_e = jnp.tile(jnp.eye(KER, dtype=f32), (N_ELEC, 1)).astype(bf16)
    sel_n = (jnp.tile(jnp.eye(KER, dtype=f32), (N_ATOMS, 1))
             * nuc_embed.reshape(LN, 1)).astype(bf16)

    wh_t = jnp.tile(Wh[1], (1, N_ELEC)).astype(bf16)
    bh_t = jnp.tile(bh[1], (N_ELEC,)).reshape(1, LE)

    wg_b = Wg.astype(bf16)
    wo_row = Wo.reshape(1, EMBED)
    bo2 = bo.reshape(1, 1)

    grid = (BATCH // B_BLK,)

    def whole(shape):
        nd = len(shape)
        return pl.BlockSpec(shape, lambda i: (0,) * nd)

    out = pl.pallas_call(
        _jastrow_kernel,
        grid=grid,
        in_specs=[
            pl.BlockSpec((B_BLK * N_ELEC, N_NBR * BASIS), lambda i: (i, 0)),
            whole(w_cat.shape),
            whole(b_cat.shape),
            whole(m0h.shape),
            whole(mask.shape),
            whole(diag.shape),
            whole(wh_t.shape),
            whole(bh_t.shape),
            whole(sel_e.shape),
            whole(sel_n.shape),
            whole(wg_b.shape),
            whole(bg.shape),
            whole(wo_row.shape),
            whole(x0.shape),
            whole(bo2.shape),
        ],
        out_specs=pl.BlockSpec((B_BLK, 1), lambda i: (i, 0)),
        out_shape=jax.ShapeDtypeStruct((BATCH, 1), f32),
    )(xs_flat, w_cat, b_cat, m0h, mask, diag, wh_t, bh_t, sel_e, sel_n, wg_b,
      bg, wo_row, x0, bo2)
    return out.reshape(BATCH)


# speedup vs baseline: 12.8378x; 2.7266x over previous
import jax
import jax.numpy as jnp
from jax.experimental import pallas as pl

N_UP = 16
N_DOWN = 16
N_ELEC = 32
N_ATOMS = 16
N_NBR = N_ELEC + N_ATOMS
BASIS = 4
KER = 8
EMBED = 16
LAYERS = 2
BATCH = 4096
LE = N_ELEC * KER
LN = N_ATOMS * KER

B_BLK = 128


def _jastrow_kernel(xs_ref, i48_ref, wcat_ref, bcat_ref, m0h_ref, mask_ref,
                    diag_ref, wht_ref, bht_ref, sele_ref, seln_ref, wg_ref,
                    bg_ref, wo_ref, x0_ref, bo_ref, out_ref):
    B = B_BLK
    R = B * N_ELEC
    bf16 = jnp.bfloat16
    f32 = jnp.float32

    Xn = xs_ref[...]
    i48 = i48_ref[...]
    tdn = (((0,), (0,)), ((), ()))
    rows = []
    for i_ in range(N_ELEC):
        cols = []
        for f in range(BASIS):
            A = Xn[i_, :, f, :].astype(bf16)
            cols.append(
                jax.lax.dot_general(A, i48, tdn,
                                    preferred_element_type=f32))
        rows.append(jnp.concatenate(cols, axis=1))
    X = jnp.concatenate(rows, axis=0).astype(bf16)

    P = jnp.tanh(
        jax.lax.dot(X, wcat_ref[...], preferred_element_type=f32)
        + bcat_ref[...])
    P0e = P[:, :LE]
    P0n = P[:, LE:LE + LN]
    P1e = P[:, LE + LN:2 * LE + LN]
    P1n = P[:, 2 * LE + LN:]

    sele = sele_ref[...]
    seln = seln_ref[...]

    M0 = (P0e.reshape(N_ELEC, B, LE) * m0h_ref[...][:, None, :]).reshape(R, LE)
    z0 = (jax.lax.dot(M0.astype(bf16), sele, preferred_element_type=f32)
          + jax.lax.dot(P0n.astype(bf16), seln, preferred_element_type=f32))
    g0 = jnp.tanh(
        jax.lax.dot(z0.astype(bf16), wg_ref[0], preferred_element_type=f32)
        + bg_ref[0][None, :])
    x1 = jnp.broadcast_to(x0_ref[...][:, None, :],
                          (N_ELEC, B, EMBED)).reshape(R, EMBED) + g0

    ph = (jax.lax.dot(x1.astype(bf16), wht_ref[...], preferred_element_type=f32)
          + bht_ref[...])
    hb = jnp.tanh(
        (ph.reshape(N_ELEC, B, LE) * diag_ref[...][:, None, :]).sum(axis=0))
    M1 = (P1e.reshape(N_ELEC, B, LE) * mask_ref[...][:, None, :]
          * hb[None, :, :]).reshape(R, LE)
    z1 = (jax.lax.dot(M1.astype(bf16), sele, preferred_element_type=f32)
          + jax.lax.dot(P1n.astype(bf16), seln, preferred_element_type=f32))
    x2 = x1 + jnp.tanh(
        jax.lax.dot(z1.astype(bf16), wg_ref[1], preferred_element_type=f32)
        + bg_ref[1][None, :])

    t = (x2 * wo_ref[...]).sum(axis=1, keepdims=True)
    out_ref[...] = (t.reshape(N_ELEC, B, 1).sum(axis=0)
                    + jnp.float32(N_ELEC) * bo_ref[0, 0])


@jax.jit
def kernel(xs, elec_embed, nuc_embed, Ww_e, bw_e, Ww_n, bw_n, Wh, bh, Wg, bg,
           Wo, bo):
    f32 = jnp.float32
    bf16 = jnp.bfloat16

    xs_n = xs.transpose(1, 2, 3, 0)
    i48 = jnp.eye(N_NBR, dtype=jnp.bfloat16)

    eye48 = jnp.eye(N_NBR, dtype=f32)
    w_sel = jnp.concatenate(
        [jnp.broadcast_to(Ww_e[:, None], (LAYERS, N_ELEC, BASIS, KER)),
         jnp.broadcast_to(Ww_n[:, None], (LAYERS, N_ATOMS, BASIS, KER))],
        axis=1)
    w_big = jnp.einsum('jJ,ljfk->ljfJk', eye48, w_sel).reshape(
        LAYERS, N_NBR * BASIS, N_NBR * KER)
    w_cat = jnp.concatenate([w_big[0], w_big[1]], axis=1).reshape(
        N_NBR, BASIS, 2 * N_NBR * KER).transpose(1, 0, 2).reshape(
        N_NBR * BASIS, 2 * N_NBR * KER).astype(bf16)
    b_big = jnp.concatenate(
        [jnp.broadcast_to(bw_e[:, None], (LAYERS, N_ELEC, KER)),
         jnp.broadcast_to(bw_n[:, None], (LAYERS, N_ATOMS, KER))],
        axis=1).reshape(LAYERS, N_NBR * KER)
    b_cat = jnp.concatenate([b_big[0], b_big[1]]).reshape(1, 2 * N_NBR * KER)

    eye_jk = jnp.repeat(jnp.eye(N_ELEC, dtype=f32), KER, axis=1)
    mask = 1.0 - eye_jk
    diag = eye_jk

    x0 = jnp.concatenate([
        jnp.broadcast_to(elec_embed[0][None], (N_UP, EMBED)),
        jnp.broadcast_to(elec_embed[1][None], (N_DOWN, EMBED)),
    ], axis=0)
    h0 = jnp.tanh(x0 @ Wh[0] + bh[0][None, :]).reshape(1, LE)
    m0h = mask * h0

    sel_e = jnp.tile(jnp.eye(KER, dtype=f32), (N_ELEC, 1)).astype(bf16)
    sel_n = (jnp.tile(jnp.eye(KER, dtype=f32), (N_ATOMS, 1))
             * nuc_embed.reshape(LN, 1)).astype(bf16)

    wh_t = jnp.tile(Wh[1], (1, N_ELEC)).astype(bf16)
    bh_t = jnp.tile(bh[1], (N_ELEC,)).reshape(1, LE)

    wg_b = Wg.astype(bf16)
    wo_row = Wo.reshape(1, EMBED)
    bo2 = bo.reshape(1, 1)

    grid = (BATCH // B_BLK,)

    def whole(shape):
        nd = len(shape)
        return pl.BlockSpec(shape, lambda i: (0,) * nd)

    out = pl.pallas_call(
        _jastrow_kernel,
        grid=grid,
        in_specs=[
            pl.BlockSpec((N_ELEC, N_NBR, BASIS, B_BLK),
                         lambda i: (0, 0, 0, i)),
            whole(i48.shape),
            whole(w_cat.shape),
            whole(b_cat.shape),
            whole(m0h.shape),
            whole(mask.shape),
            whole(diag.shape),
            whole(wh_t.shape),
            whole(bh_t.shape),
            whole(sel_e.shape),
            whole(sel_n.shape),
            whole(wg_b.shape),
            whole(bg.shape),
            whole(wo_row.shape),
            whole(x0.shape),
            whole(bo2.shape),
        ],
        out_specs=pl.BlockSpec((B_BLK, 1), lambda i: (i, 0)),
        out_shape=jax.ShapeDtypeStruct((BATCH, 1), f32),
    )(xs_n, i48, w_cat, b_cat, m0h, mask, diag, wh_t, bh_t, sel_e, sel_n,
      wg_b, bg, wo_row, x0, bo2)
    return out.reshape(BATCH)


# speedup vs baseline: 13.0599x; 1.0173x over previous
import jax
import jax.numpy as jnp
from jax.experimental import pallas as pl

N_UP = 16
N_DOWN = 16
N_ELEC = 32
N_ATOMS = 16
N_NBR = N_ELEC + N_ATOMS
BASIS = 4
KER = 8
EMBED = 16
LAYERS = 2
BATCH = 4096
LE = N_ELEC * KER
LN = N_ATOMS * KER

B_BLK = 128


def _jastrow_kernel(xs_ref, i48_ref, wcat_ref, bcat_ref, m0h_ref, mask_ref,
                    diag_ref, wht_ref, bht_ref, sele_ref, seln_ref, wg_ref,
                    bg_ref, wo_ref, x0_ref, bo_ref, out_ref):
    B = B_BLK
    R = B * N_ELEC
    bf16 = jnp.bfloat16
    f32 = jnp.float32

    Xn = xs_ref[...]
    i48 = i48_ref[...]
    tdn = (((0,), (0,)), ((), ()))
    rows = []
    for i_ in range(N_ELEC):
        cols = []
        for f in range(BASIS):
            A = Xn[i_, :, f, :].astype(bf16)
            cols.append(
                jax.lax.dot_general(A, i48, tdn,
                                    preferred_element_type=f32))
        rows.append(jnp.concatenate(cols, axis=1))
    X = jnp.concatenate(rows, axis=0).astype(bf16)

    P = jnp.tanh(
        jax.lax.dot(X, wcat_ref[...], preferred_element_type=f32)
        + bcat_ref[...]).astype(bf16)
    P0e = P[:, :LE]
    P0n = P[:, LE:LE + LN]
    P1e = P[:, LE + LN:2 * LE + LN]
    P1n = P[:, 2 * LE + LN:]

    sele = sele_ref[...]
    seln = seln_ref[...]

    M0 = (P0e.reshape(N_ELEC, B, LE) * m0h_ref[...][:, None, :]).reshape(R, LE)
    z0 = (jax.lax.dot(M0, sele, preferred_element_type=f32)
          + jax.lax.dot(P0n, seln, preferred_element_type=f32))
    g0 = jnp.tanh(
        jax.lax.dot(z0.astype(bf16), wg_ref[0], preferred_element_type=f32)
        + bg_ref[0][None, :])
    x1 = jnp.broadcast_to(x0_ref[...][:, None, :],
                          (N_ELEC, B, EMBED)).reshape(R, EMBED) + g0

    ph = (jax.lax.dot(x1.astype(bf16), wht_ref[...], preferred_element_type=f32)
          + bht_ref[...])
    hb = jnp.tanh(
        (ph.reshape(N_ELEC, B, LE) * diag_ref[...][:, None, :]).sum(axis=0)
    ).astype(bf16)
    M1 = (P1e.reshape(N_ELEC, B, LE) * mask_ref[...][:, None, :]
          * hb[None, :, :]).reshape(R, LE)
    z1 = (jax.lax.dot(M1, sele, preferred_element_type=f32)
          + jax.lax.dot(P1n, seln, preferred_element_type=f32))
    x2 = x1 + jnp.tanh(
        jax.lax.dot(z1.astype(bf16), wg_ref[1], preferred_element_type=f32)
        + bg_ref[1][None, :])

    t = (x2 * wo_ref[...]).sum(axis=1, keepdims=True)
    out_ref[...] = (t.reshape(N_ELEC, B, 1).sum(axis=0)
                    + jnp.float32(N_ELEC) * bo_ref[0, 0])


@jax.jit
def kernel(xs, elec_embed, nuc_embed, Ww_e, bw_e, Ww_n, bw_n, Wh, bh, Wg, bg,
           Wo, bo):
    f32 = jnp.float32
    bf16 = jnp.bfloat16

    xs_n = xs.transpose(1, 2, 3, 0)
    i48 = jnp.eye(N_NBR, dtype=jnp.bfloat16)

    eye48 = jnp.eye(N_NBR, dtype=f32)
    w_sel = jnp.concatenate(
        [jnp.broadcast_to(Ww_e[:, None], (LAYERS, N_ELEC, BASIS, KER)),
         jnp.broadcast_to(Ww_n[:, None], (LAYERS, N_ATOMS, BASIS, KER))],
        axis=1)
    w_big = jnp.einsum('jJ,ljfk->ljfJk', eye48, w_sel).reshape(
        LAYERS, N_NBR * BASIS, N_NBR * KER)
    w_cat = jnp.concatenate([w_big[0], w_big[1]], axis=1).reshape(
        N_NBR, BASIS, 2 * N_NBR * KER).transpose(1, 0, 2).reshape(
        N_NBR * BASIS, 2 * N_NBR * KER).astype(bf16)
    b_big = jnp.concatenate(
        [jnp.broadcast_to(bw_e[:, None], (LAYERS, N_ELEC, KER)),
         jnp.broadcast_to(bw_n[:, None], (LAYERS, N_ATOMS, KER))],
        axis=1).reshape(LAYERS, N_NBR * KER)
    b_cat = jnp.concatenate([b_big[0], b_big[1]]).reshape(1, 2 * N_NBR * KER)

    eye_jk = jnp.repeat(jnp.eye(N_ELEC, dtype=f32), KER, axis=1)
    mask = 1.0 - eye_jk
    diag = eye_jk

    x0 = jnp.concatenate([
        jnp.broadcast_to(elec_embed[0][None], (N_UP, EMBED)),
        jnp.broadcast_to(elec_embed[1][None], (N_DOWN, EMBED)),
    ], axis=0)
    h0 = jnp.tanh(x0 @ Wh[0] + bh[0][None, :]).reshape(1, LE)
    m0h = (mask * h0).astype(bf16)
    mask = mask.astype(bf16)

    sel_e = jnp.tile(jnp.eye(KER, dtype=f32), (N_ELEC, 1)).astype(bf16)
    sel_n = (jnp.tile(jnp.eye(KER, dtype=f32), (N_ATOMS, 1))
             * nuc_embed.reshape(LN, 1)).astype(bf16)

    wh_t = jnp.tile(Wh[1], (1, N_ELEC)).astype(bf16)
    bh_t = jnp.tile(bh[1], (N_ELEC,)).reshape(1, LE)

    wg_b = Wg.astype(bf16)
    wo_row = Wo.reshape(1, EMBED)
    bo2 = bo.reshape(1, 1)

    grid = (BATCH // B_BLK,)

    def whole(shape):
        nd = len(shape)
        return pl.BlockSpec(shape, lambda i: (0,) * nd)

    out = pl.pallas_call(
        _jastrow_kernel,
        grid=grid,
        in_specs=[
            pl.BlockSpec((N_ELEC, N_NBR, BASIS, B_BLK),
                         lambda i: (0, 0, 0, i)),
            whole(i48.shape),
            whole(w_cat.shape),
            whole(b_cat.shape),
            whole(m0h.shape),
            whole(mask.shape),
            whole(diag.shape),
            whole(wh_t.shape),
            whole(bh_t.shape),
            whole(sel_e.shape),
            whole(sel_n.shape),
            whole(wg_b.shape),
            whole(bg.shape),
            whole(wo_row.shape),
            whole(x0.shape),
            whole(bo2.shape),
        ],
        out_specs=pl.BlockSpec((B_BLK, 1), lambda i: (i, 0)),
        out_shape=jax.ShapeDtypeStruct((BATCH, 1), f32),
    )(xs_n, i48, w_cat, b_cat, m0h, mask, diag, wh_t, bh_t, sel_e, sel_n,
      wg_b, bg, wo_row, x0, bo2)
    return out.reshape(BATCH)


# speedup vs baseline: 13.8922x; 1.0637x over previous
import jax
import jax.numpy as jnp
from jax.experimental import pallas as pl

N_UP = 16
N_DOWN = 16
N_ELEC = 32
N_ATOMS = 16
N_NBR = N_ELEC + N_ATOMS
BASIS = 4
KER = 8
EMBED = 16
LAYERS = 2
BATCH = 4096
LE = N_ELEC * KER
LN = N_ATOMS * KER

B_BLK = 128


def _jastrow_kernel(xs_ref, i48_ref, wcat_ref, bcat_ref, m0h_ref, mask_ref,
                    diag_ref, wht_ref, bht_ref, sele_ref, seln_ref, wg_ref,
                    bg_ref, wo_ref, x0_ref, bo_ref, out_ref):
    B = B_BLK
    R = B * N_ELEC
    bf16 = jnp.bfloat16
    f32 = jnp.float32

    Xn = xs_ref[...]
    i192 = i48_ref[...]
    tdn = (((0,), (0,)), ((), ()))
    rows = []
    for i_ in range(N_ELEC):
        A = jnp.concatenate(
            [Xn[i_, :, f, :] for f in range(BASIS)], axis=0).astype(bf16)
        rows.append(
            jax.lax.dot_general(A, i192, tdn,
                                preferred_element_type=f32))
    X = jnp.concatenate(rows, axis=0).astype(bf16)

    P = jnp.tanh(
        jax.lax.dot(X, wcat_ref[...], preferred_element_type=f32)
        + bcat_ref[...]).astype(bf16)
    P0e = P[:, :LE]
    P0n = P[:, LE:LE + LN]
    P1e = P[:, LE + LN:2 * LE + LN]
    P1n = P[:, 2 * LE + LN:]

    sele = sele_ref[...]
    seln = seln_ref[...]

    M0 = (P0e.reshape(N_ELEC, B, LE) * m0h_ref[...][:, None, :]).reshape(R, LE)
    z0 = (jax.lax.dot(M0, sele, preferred_element_type=f32)
          + jax.lax.dot(P0n, seln, preferred_element_type=f32))
    g0 = jnp.tanh(
        jax.lax.dot(z0.astype(bf16), wg_ref[0], preferred_element_type=f32)
        + bg_ref[0][None, :])
    x1 = jnp.broadcast_to(x0_ref[...][:, None, :],
                          (N_ELEC, B, EMBED)).reshape(R, EMBED) + g0

    ph = (jax.lax.dot(x1.astype(bf16), wht_ref[...], preferred_element_type=f32)
          + bht_ref[...])
    hb = jnp.tanh(
        (ph.reshape(N_ELEC, B, LE) * diag_ref[...][:, None, :]).sum(axis=0)
    ).astype(bf16)
    M1 = (P1e.reshape(N_ELEC, B, LE) * mask_ref[...][:, None, :]
          * hb[None, :, :]).reshape(R, LE)
    z1 = (jax.lax.dot(M1, sele, preferred_element_type=f32)
          + jax.lax.dot(P1n, seln, preferred_element_type=f32))
    x2 = x1 + jnp.tanh(
        jax.lax.dot(z1.astype(bf16), wg_ref[1], preferred_element_type=f32)
        + bg_ref[1][None, :])

    t = (x2 * wo_ref[...]).sum(axis=1, keepdims=True)
    out_ref[...] = (t.reshape(N_ELEC, B, 1).sum(axis=0)
                    + jnp.float32(N_ELEC) * bo_ref[0, 0])


@jax.jit
def kernel(xs, elec_embed, nuc_embed, Ww_e, bw_e, Ww_n, bw_n, Wh, bh, Wg, bg,
           Wo, bo):
    f32 = jnp.float32
    bf16 = jnp.bfloat16

    xs_n = xs.transpose(1, 2, 3, 0)
    i48 = jnp.eye(N_NBR * BASIS, dtype=jnp.bfloat16)

    eye48 = jnp.eye(N_NBR, dtype=f32)
    w_sel = jnp.concatenate(
        [jnp.broadcast_to(Ww_e[:, None], (LAYERS, N_ELEC, BASIS, KER)),
         jnp.broadcast_to(Ww_n[:, None], (LAYERS, N_ATOMS, BASIS, KER))],
        axis=1)
    w_big = jnp.einsum('jJ,ljfk->ljfJk', eye48, w_sel).reshape(
        LAYERS, N_NBR * BASIS, N_NBR * KER)
    w_cat = jnp.concatenate([w_big[0], w_big[1]], axis=1).reshape(
        N_NBR, BASIS, 2 * N_NBR * KER).transpose(1, 0, 2).reshape(
        N_NBR * BASIS, 2 * N_NBR * KER).astype(bf16)
    b_big = jnp.concatenate(
        [jnp.broadcast_to(bw_e[:, None], (LAYERS, N_ELEC, KER)),
         jnp.broadcast_to(bw_n[:, None], (LAYERS, N_ATOMS, KER))],
        axis=1).reshape(LAYERS, N_NBR * KER)
    b_cat = jnp.concatenate([b_big[0], b_big[1]]).reshape(1, 2 * N_NBR * KER)

    eye_jk = jnp.repeat(jnp.eye(N_ELEC, dtype=f32), KER, axis=1)
    mask = 1.0 - eye_jk
    diag = eye_jk

    x0 = jnp.concatenate([
        jnp.broadcast_to(elec_embed[0][None], (N_UP, EMBED)),
        jnp.broadcast_to(elec_embed[1][None], (N_DOWN, EMBED)),
    ], axis=0)
    h0 = jnp.tanh(x0 @ Wh[0] + bh[0][None, :]).reshape(1, LE)
    m0h = (mask * h0).astype(bf16)
    mask = mask.astype(bf16)

    sel_e = jnp.tile(jnp.eye(KER, dtype=f32), (N_ELEC, 1)).astype(bf16)
    sel_n = (jnp.tile(jnp.eye(KER, dtype=f32), (N_ATOMS, 1))
             * nuc_embed.reshape(LN, 1)).astype(bf16)

    wh_t = jnp.tile(Wh[1], (1, N_ELEC)).astype(bf16)
    bh_t = jnp.tile(bh[1], (N_ELEC,)).reshape(1, LE)

    wg_b = Wg.astype(bf16)
    wo_row = Wo.reshape(1, EMBED)
    bo2 = bo.reshape(1, 1)

    grid = (BATCH // B_BLK,)

    def whole(shape):
        nd = len(shape)
        return pl.BlockSpec(shape, lambda i: (0,) * nd)

    out = pl.pallas_call(
        _jastrow_kernel,
        grid=grid,
        in_specs=[
            pl.BlockSpec((N_ELEC, N_NBR, BASIS, B_BLK),
                         lambda i: (0, 0, 0, i)),
            whole(i48.shape),
            whole(w_cat.shape),
            whole(b_cat.shape),
            whole(m0h.shape),
            whole(mask.shape),
            whole(diag.shape),
            whole(wh_t.shape),
            whole(bh_t.shape),
            whole(sel_e.shape),
            whole(sel_n.shape),
            whole(wg_b.shape),
            whole(bg.shape),
            whole(wo_row.shape),
            whole(x0.shape),
            whole(bo2.shape),
        ],
        out_specs=pl.BlockSpec((B_BLK, 1), lambda i: (i, 0)),
        out_shape=jax.ShapeDtypeStruct((BATCH, 1), f32),
    )(xs_n, i48, w_cat, b_cat, m0h, mask, diag, wh_t, bh_t, sel_e, sel_n,
      wg_b, bg, wo_row, x0, bo2)
    return out.reshape(BATCH)
